# baseline (speedup 1.0000x reference)
import functools

import jax
import jax.numpy as jnp
from jax import lax
from jax.experimental import pallas as pl
from jax.experimental.pallas import tpu as pltpu
from jax.experimental.pallas import tpu_sc as plsc

NC = 2
NS = 16
NW = NC * NS
CHUNK = 128
DEPTH = 4
EPS = 1e-5


def _deg_kernel(E_pad, N_pad):
    epw = E_pad // NW
    n_chunks = epw // CHUNK
    rpt = N_pad // NS
    mesh = plsc.VectorSubcoreMesh(core_axis_name="c", subcore_axis_name="s")

    @functools.partial(
        pl.kernel,
        out_type=jax.ShapeDtypeStruct((NC * N_pad,), jnp.float32),
        mesh=mesh,
        scratch_types=[
            pltpu.VMEM((n_chunks, 1, CHUNK), jnp.int32),
            pltpu.VMEM((CHUNK,), jnp.float32),
            pltpu.VMEM_SHARED((N_pad,), jnp.float32),
            pltpu.SemaphoreType.DMA,
        ],
    )
    def body(dst_hbm, ones_hbm, zvec_hbm, out_hbm, idx_v, ones_v, acc_sh, sem):
        c = lax.axis_index("c")
        s = lax.axis_index("s")
        r0 = s * rpt
        w = c * NS + s
        pltpu.sync_copy(zvec_hbm, acc_sh.at[pl.ds(r0, rpt)])
        pltpu.sync_copy(ones_hbm, ones_v)
        pltpu.sync_copy(dst_hbm.at[w], idx_v)
        plsc.subcore_barrier()

        def step(k, carry):
            pl.when(k >= DEPTH)(
                lambda: pltpu.make_async_copy(
                    ones_v, acc_sh.at[idx_v.at[k - DEPTH, 0]], sem).wait())
            pltpu.async_copy(ones_v, acc_sh.at[idx_v.at[k, 0]], sem, add=True)
            return carry

        lax.fori_loop(0, n_chunks, step, 0)
        for k in range(n_chunks - DEPTH, n_chunks):
            pltpu.make_async_copy(ones_v, acc_sh.at[idx_v.at[k, 0]], sem).wait()
        plsc.subcore_barrier()
        pltpu.sync_copy(acc_sh.at[pl.ds(r0, rpt)],
                        out_hbm.at[pl.ds(c * N_pad + r0, rpt)])

    return body


def _agg_kernel(E_pad, N_pad, D):
    k = E_pad // NW // CHUNK
    rpt = N_pad // NS
    mesh = plsc.VectorSubcoreMesh(core_axis_name="c", subcore_axis_name="s")

    @functools.partial(
        pl.kernel,
        out_type=jax.ShapeDtypeStruct((NC, N_pad, D), jnp.float32),
        mesh=mesh,
        scratch_types=[
            pltpu.VMEM((DEPTH, 2, 1, CHUNK), jnp.int32),
            pltpu.VMEM((2, CHUNK, D), jnp.float32),
            pltpu.VMEM_SHARED((N_pad, D), jnp.float32),
            pltpu.SemaphoreType.DMA,
            pltpu.SemaphoreType.DMA,
            pltpu.SemaphoreType.DMA,
        ],
    )
    def body(edges_hbm, hp_hbm, out_hbm,
             ebuf, rows, acc_sh, sem_i, sem_g, sem_s):
        c = lax.axis_index("c")
        s = lax.axis_index("s")
        r0 = s * rpt
        base = (c * NS + s) * k

        def zrow(r, carry):
            for q in range(D // 16):
                rows[0, r, pl.ds(q * 16, 16)] = jnp.zeros((16,), jnp.float32)
            return carry

        lax.fori_loop(0, CHUNK, zrow, 0)
        for q in range(rpt // CHUNK):
            pltpu.sync_copy(rows.at[0],
                            acc_sh.at[pl.ds(r0 + q * CHUNK, CHUNK)])
        plsc.subcore_barrier()

        def idx_load(t, j):
            pltpu.async_copy(edges_hbm.at[base + t], ebuf.at[j], sem_i)

        def wait_idx(t, j):
            pltpu.make_async_copy(edges_hbm.at[base + t], ebuf.at[j],
                                  sem_i).wait()

        def gather(t, j, b):
            pltpu.async_copy(hp_hbm.at[ebuf.at[j, 0, 0]], rows.at[b], sem_g)

        def wait_gather(t, j, b):
            pltpu.make_async_copy(
                hp_hbm.at[ebuf.at[j, 0, 0]], rows.at[b], sem_g).wait()

        def scatter(t, j, b):
            pltpu.async_copy(rows.at[b], acc_sh.at[ebuf.at[j, 1, 0]], sem_s,
                             add=True)

        def wait_scatter(t, j, b):
            pltpu.make_async_copy(
                rows.at[b], acc_sh.at[ebuf.at[j, 1, 0]], sem_s).wait()

        for t in range(3):
            idx_load(t, t)
        wait_idx(0, 0)
        gather(0, 0, 0)

        def step(i, carry):
            t0 = i * DEPTH
            for u in range(DEPTH):
                tt = t0 + u
                j, b = u, u % 2
                jn, bn = (u + 1) % DEPTH, (u + 1) % 2
                jp, bp = (u + 3) % DEPTH, (u + 1) % 2

                wait_gather(tt, j, b)
                scatter(tt, j, b)
                pl.when(tt >= 1)(
                    lambda tt=tt, jp=jp, bp=bp: wait_scatter(tt - 1, jp, bp))

                def _next(tt=tt, jn=jn, bn=bn):
                    wait_idx(tt + 1, jn)
                    gather(tt + 1, jn, bn)

                pl.when(tt + 1 < k)(_next)
                pl.when(tt + 3 < k)(
                    lambda tt=tt, u=u: idx_load(tt + 3, (u + 3) % DEPTH))
            return carry

        lax.fori_loop(0, k // DEPTH, step, 0)
        wait_scatter(k - 1, (k - 1) % DEPTH, (k - 1) % 2)
        plsc.subcore_barrier()
        pl.when(s == 0)(
            lambda: pltpu.sync_copy(acc_sh, out_hbm.at[c]))

    return body


def _hprime(x_pad, Wm, deg_parts):
    N_pad, D = x_pad.shape

    def body(x_ref, w_ref, dp_ref, o_ref):
        deg = dp_ref[0, :] + dp_ref[1, :] + 1.0
        dinv = lax.rsqrt(deg)
        h = jnp.dot(x_ref[...], w_ref[...], preferred_element_type=jnp.float32)
        o_ref[...] = h * dinv[:, None]

    return pl.pallas_call(
        body,
        out_shape=jax.ShapeDtypeStruct((N_pad, D), jnp.float32),
    )(x_pad, Wm, deg_parts)


def _finalize(acc, hp, deg_parts, b2, g2, be2, n):
    _, N_pad, D = acc.shape

    def body(acc_ref, hp_ref, dp_ref, b_ref, g_ref, be_ref, o_ref):
        deg = dp_ref[0, :n] + dp_ref[1, :n] + 1.0
        dinv = lax.rsqrt(deg)
        tot = acc_ref[0, :n, :] + acc_ref[1, :n, :] + hp_ref[:n, :]
        pre = tot * dinv[:, None] + b_ref[...]
        mean = jnp.mean(pre, axis=0, keepdims=True)
        var = jnp.mean((pre - mean) ** 2, axis=0, keepdims=True)
        out = (pre - mean) * lax.rsqrt(var + EPS)
        o_ref[...] = jnp.maximum(out * g_ref[...] + be_ref[...], 0.0)

    return pl.pallas_call(
        body,
        out_shape=jax.ShapeDtypeStruct((n, D), jnp.float32),
    )(acc, hp, deg_parts, b2, g2, be2)


def kernel(x, edge_index, W, b, gamma, beta):
    N, D = x.shape
    E = edge_index.shape[1]
    src = edge_index[0].astype(jnp.int32)
    dst = edge_index[1].astype(jnp.int32)

    grain = NW * CHUNK * DEPTH
    E_pad = -(-E // grain) * grain
    N_pad = -(-(N + 1) // (NS * 128)) * (NS * 128)
    epw = E_pad // NW

    n_fill = E_pad - E
    fill = jnp.arange(n_fill, dtype=jnp.int32)
    src_pad = jnp.concatenate([src, fill % N])
    dst_pad = jnp.concatenate([dst, N + fill % (N_pad - N)])
    x_pad = jnp.pad(x, ((0, N_pad - N), (0, 0)))

    dst_l = dst_pad.reshape(NW, epw // CHUNK, 1, CHUNK)
    edges_l = (jnp.stack([src_pad, dst_pad])
               .reshape(2, E_pad // CHUNK, 1, CHUNK)
               .transpose(1, 0, 2, 3))

    ones_c = jnp.ones((CHUNK,), jnp.float32)
    zvec = jnp.zeros((N_pad // NS,), jnp.float32)

    deg_parts = _deg_kernel(E_pad, N_pad)(dst_l, ones_c, zvec).reshape(NC, N_pad)
    hp = _hprime(x_pad, W, deg_parts)
    acc = _agg_kernel(E_pad, N_pad, D)(edges_l, hp)
    return _finalize(acc, hp, deg_parts,
                     b.reshape(1, D), gamma.reshape(1, D), beta.reshape(1, D), N)


# speedup vs baseline: 1.1384x; 1.1384x over previous
import functools

import jax
import jax.numpy as jnp
from jax import lax
from jax.experimental import pallas as pl
from jax.experimental.pallas import tpu as pltpu
from jax.experimental.pallas import tpu_sc as plsc

NC = 2
NS = 16
NW = NC * NS
CHUNK = 128
DEPTH = 4
EPS = 1e-5


def _deg_kernel(E_pad, N_pad):
    epw = E_pad // NW
    n_chunks = epw // CHUNK
    rpt = N_pad // NS
    mesh = plsc.VectorSubcoreMesh(core_axis_name="c", subcore_axis_name="s")

    @functools.partial(
        pl.kernel,
        out_type=jax.ShapeDtypeStruct((NC * N_pad,), jnp.float32),
        mesh=mesh,
        scratch_types=[
            pltpu.VMEM((n_chunks, 1, CHUNK), jnp.int32),
            pltpu.VMEM((CHUNK,), jnp.float32),
            pltpu.VMEM_SHARED((N_pad,), jnp.float32),
            pltpu.SemaphoreType.DMA,
        ],
    )
    def body(dst_hbm, ones_hbm, zvec_hbm, out_hbm, idx_v, ones_v, acc_sh, sem):
        c = lax.axis_index("c")
        s = lax.axis_index("s")
        r0 = s * rpt
        w = c * NS + s
        pltpu.sync_copy(zvec_hbm, acc_sh.at[pl.ds(r0, rpt)])
        pltpu.sync_copy(ones_hbm, ones_v)
        pltpu.sync_copy(dst_hbm.at[w], idx_v)
        plsc.subcore_barrier()

        def step(k, carry):
            pl.when(k >= DEPTH)(
                lambda: pltpu.make_async_copy(
                    ones_v, acc_sh.at[idx_v.at[k - DEPTH, 0]], sem).wait())
            pltpu.async_copy(ones_v, acc_sh.at[idx_v.at[k, 0]], sem, add=True)
            return carry

        lax.fori_loop(0, n_chunks, step, 0)
        for k in range(n_chunks - DEPTH, n_chunks):
            pltpu.make_async_copy(ones_v, acc_sh.at[idx_v.at[k, 0]], sem).wait()
        plsc.subcore_barrier()
        pltpu.sync_copy(acc_sh.at[pl.ds(r0, rpt)],
                        out_hbm.at[pl.ds(c * N_pad + r0, rpt)])

    return body


def _agg_kernel(E_pad, N_pad, D):
    k = E_pad // NW // CHUNK
    rpt = N_pad // NS
    mesh = plsc.VectorSubcoreMesh(core_axis_name="c", subcore_axis_name="s")

    @functools.partial(
        pl.kernel,
        out_type=jax.ShapeDtypeStruct((NC, N_pad, D), jnp.float32),
        mesh=mesh,
        scratch_types=[
            pltpu.VMEM((DEPTH, 2, 1, CHUNK), jnp.int32),
            pltpu.VMEM((2, CHUNK, D), jnp.float32),
            pltpu.VMEM_SHARED((N_pad, D), jnp.float32),
            pltpu.SemaphoreType.DMA,
            pltpu.SemaphoreType.DMA,
        ],
    )
    def body(edges_hbm, hp_hbm, out_hbm,
             ebuf, rows, acc_sh, sem_i, sem_g):
        c = lax.axis_index("c")
        s = lax.axis_index("s")
        r0 = s * rpt
        base = (c * NS + s) * k

        def zrow(r, carry):
            for q in range(D // 16):
                rows[0, r, pl.ds(q * 16, 16)] = jnp.zeros((16,), jnp.float32)
            return carry

        lax.fori_loop(0, CHUNK, zrow, 0)
        for q in range(rpt // CHUNK):
            pltpu.sync_copy(rows.at[0],
                            acc_sh.at[pl.ds(r0 + q * CHUNK, CHUNK)])
        plsc.subcore_barrier()

        def idx_load(t, j):
            pltpu.async_copy(edges_hbm.at[base + t], ebuf.at[j], sem_i)

        def wait_idx(t, j):
            pltpu.make_async_copy(edges_hbm.at[base + t], ebuf.at[j],
                                  sem_i).wait()

        def gather(t, j, b):
            pltpu.async_copy(hp_hbm.at[ebuf.at[j, 0, 0]], rows.at[b], sem_g)

        def wait_gather(t, j, b):
            pltpu.make_async_copy(
                hp_hbm.at[ebuf.at[j, 0, 0]], rows.at[b], sem_g).wait()

        def scatter(t, j, b):
            pltpu.sync_copy(rows.at[b], acc_sh.at[ebuf.at[j, 1, 0]], add=True)

        for t in range(3):
            idx_load(t, t)
        wait_idx(0, 0)
        gather(0, 0, 0)

        def step(i, carry):
            t0 = i * DEPTH
            for u in range(DEPTH):
                tt = t0 + u
                j, b = u, u % 2
                jn, bn = (u + 1) % DEPTH, (u + 1) % 2

                def _next(tt=tt, jn=jn, bn=bn):
                    wait_idx(tt + 1, jn)
                    gather(tt + 1, jn, bn)

                pl.when(tt + 1 < k)(_next)
                pl.when(tt + 3 < k)(
                    lambda tt=tt, u=u: idx_load(tt + 3, (u + 3) % DEPTH))
                wait_gather(tt, j, b)
                scatter(tt, j, b)
            return carry

        lax.fori_loop(0, k // DEPTH, step, 0)
        plsc.subcore_barrier()
        pl.when(s == 0)(
            lambda: pltpu.sync_copy(acc_sh, out_hbm.at[c]))

    return body


def _hprime(x_pad, Wm, deg_parts):
    N_pad, D = x_pad.shape

    def body(x_ref, w_ref, dp_ref, o_ref):
        deg = dp_ref[0, :] + dp_ref[1, :] + 1.0
        dinv = lax.rsqrt(deg)
        h = jnp.dot(x_ref[...], w_ref[...], preferred_element_type=jnp.float32)
        o_ref[...] = h * dinv[:, None]

    return pl.pallas_call(
        body,
        out_shape=jax.ShapeDtypeStruct((N_pad, D), jnp.float32),
    )(x_pad, Wm, deg_parts)


def _finalize(acc, hp, deg_parts, b2, g2, be2, n):
    _, N_pad, D = acc.shape

    def body(acc_ref, hp_ref, dp_ref, b_ref, g_ref, be_ref, o_ref):
        deg = dp_ref[0, :n] + dp_ref[1, :n] + 1.0
        dinv = lax.rsqrt(deg)
        tot = acc_ref[0, :n, :] + acc_ref[1, :n, :] + hp_ref[:n, :]
        pre = tot * dinv[:, None] + b_ref[...]
        mean = jnp.mean(pre, axis=0, keepdims=True)
        var = jnp.mean((pre - mean) ** 2, axis=0, keepdims=True)
        out = (pre - mean) * lax.rsqrt(var + EPS)
        o_ref[...] = jnp.maximum(out * g_ref[...] + be_ref[...], 0.0)

    return pl.pallas_call(
        body,
        out_shape=jax.ShapeDtypeStruct((n, D), jnp.float32),
    )(acc, hp, deg_parts, b2, g2, be2)


def kernel(x, edge_index, W, b, gamma, beta):
    N, D = x.shape
    E = edge_index.shape[1]
    src = edge_index[0].astype(jnp.int32)
    dst = edge_index[1].astype(jnp.int32)

    grain = NW * CHUNK * DEPTH
    E_pad = -(-E // grain) * grain
    N_pad = -(-(N + 1) // (NS * 128)) * (NS * 128)
    epw = E_pad // NW

    n_fill = E_pad - E
    fill = jnp.arange(n_fill, dtype=jnp.int32)
    src_pad = jnp.concatenate([src, fill % N])
    dst_pad = jnp.concatenate([dst, N + fill % (N_pad - N)])
    x_pad = jnp.pad(x, ((0, N_pad - N), (0, 0)))

    dst_l = dst_pad.reshape(NW, epw // CHUNK, 1, CHUNK)
    edges_l = (jnp.stack([src_pad, dst_pad])
               .reshape(2, E_pad // CHUNK, 1, CHUNK)
               .transpose(1, 0, 2, 3))

    ones_c = jnp.ones((CHUNK,), jnp.float32)
    zvec = jnp.zeros((N_pad // NS,), jnp.float32)

    deg_parts = _deg_kernel(E_pad, N_pad)(dst_l, ones_c, zvec).reshape(NC, N_pad)
    hp = _hprime(x_pad, W, deg_parts)
    acc = _agg_kernel(E_pad, N_pad, D)(edges_l, hp)
    return _finalize(acc, hp, deg_parts,
                     b.reshape(1, D), gamma.reshape(1, D), beta.reshape(1, D), N)


# speedup vs baseline: 1.1576x; 1.0169x over previous
import functools

import jax
import jax.numpy as jnp
from jax import lax
from jax.experimental import pallas as pl
from jax.experimental.pallas import tpu as pltpu
from jax.experimental.pallas import tpu_sc as plsc

NC = 2
NS = 16
NW = NC * NS
CHUNK = 128
DEPTH = 4
EPS = 1e-5


def _deg_kernel(E_pad, N_pad):
    epw = E_pad // NW
    n_chunks = epw // CHUNK
    rpt = N_pad // NS
    mesh = plsc.VectorSubcoreMesh(core_axis_name="c", subcore_axis_name="s")

    @functools.partial(
        pl.kernel,
        out_type=jax.ShapeDtypeStruct((NC * N_pad,), jnp.float32),
        mesh=mesh,
        scratch_types=[
            pltpu.VMEM((n_chunks, 1, CHUNK), jnp.int32),
            pltpu.VMEM((CHUNK,), jnp.float32),
            pltpu.VMEM_SHARED((N_pad,), jnp.float32),
            pltpu.SemaphoreType.DMA,
        ],
    )
    def body(dst_hbm, ones_hbm, zvec_hbm, out_hbm, idx_v, ones_v, acc_sh, sem):
        c = lax.axis_index("c")
        s = lax.axis_index("s")
        r0 = s * rpt
        w = c * NS + s
        pltpu.sync_copy(zvec_hbm, acc_sh.at[pl.ds(r0, rpt)])
        pltpu.sync_copy(ones_hbm, ones_v)
        pltpu.sync_copy(dst_hbm.at[w], idx_v)
        plsc.subcore_barrier()

        def step(k, carry):
            pl.when(k >= DEPTH)(
                lambda: pltpu.make_async_copy(
                    ones_v, acc_sh.at[idx_v.at[k - DEPTH, 0]], sem).wait())
            pltpu.async_copy(ones_v, acc_sh.at[idx_v.at[k, 0]], sem, add=True)
            return carry

        lax.fori_loop(0, n_chunks, step, 0)
        for k in range(n_chunks - DEPTH, n_chunks):
            pltpu.make_async_copy(ones_v, acc_sh.at[idx_v.at[k, 0]], sem).wait()
        plsc.subcore_barrier()
        pltpu.sync_copy(acc_sh.at[pl.ds(r0, rpt)],
                        out_hbm.at[pl.ds(c * N_pad + r0, rpt)])

    return body


def _agg_kernel(E_pad, N_pad, D):
    k = E_pad // NW // CHUNK
    rpt = N_pad // NS
    mesh = plsc.VectorSubcoreMesh(core_axis_name="c", subcore_axis_name="s")

    @functools.partial(
        pl.kernel,
        out_type=jax.ShapeDtypeStruct((NC, N_pad, D), jnp.float32),
        mesh=mesh,
        scratch_types=[
            pltpu.VMEM((DEPTH, 2, 1, CHUNK), jnp.int32),
            pltpu.VMEM((2, CHUNK, D), jnp.float32),
            pltpu.VMEM_SHARED((N_pad, D), jnp.float32),
            pltpu.SemaphoreType.DMA,
            pltpu.SemaphoreType.DMA,
        ],
    )
    def body(src_hbm, dst_hbm, hp_hbm, out_hbm,
             ebuf, rows, acc_sh, sem_i, sem_g):
        c = lax.axis_index("c")
        s = lax.axis_index("s")
        r0 = s * rpt
        base = (c * NS + s) * k

        def zrow(r, carry):
            for q in range(D // 16):
                rows[0, r, pl.ds(q * 16, 16)] = jnp.zeros((16,), jnp.float32)
            return carry

        lax.fori_loop(0, CHUNK, zrow, 0)
        for q in range(rpt // CHUNK):
            pltpu.sync_copy(rows.at[0],
                            acc_sh.at[pl.ds(r0 + q * CHUNK, CHUNK)])
        plsc.subcore_barrier()

        def idx_load(t, j):
            pltpu.async_copy(src_hbm.at[base + t], ebuf.at[j, 0], sem_i)
            pltpu.async_copy(dst_hbm.at[base + t], ebuf.at[j, 1], sem_i)

        def wait_idx(t, j):
            pltpu.make_async_copy(src_hbm.at[base + t], ebuf.at[j, 0],
                                  sem_i).wait()
            pltpu.make_async_copy(dst_hbm.at[base + t], ebuf.at[j, 1],
                                  sem_i).wait()

        def gather(t, j, b):
            pltpu.async_copy(hp_hbm.at[ebuf.at[j, 0, 0]], rows.at[b], sem_g)

        def wait_gather(t, j, b):
            pltpu.make_async_copy(
                hp_hbm.at[ebuf.at[j, 0, 0]], rows.at[b], sem_g).wait()

        def scatter(t, j, b):
            pltpu.sync_copy(rows.at[b], acc_sh.at[ebuf.at[j, 1, 0]], add=True)

        for t in range(3):
            idx_load(t, t)
        wait_idx(0, 0)
        gather(0, 0, 0)

        def step(i, carry):
            t0 = i * DEPTH
            for u in range(DEPTH):
                tt = t0 + u
                j, b = u, u % 2
                jn, bn = (u + 1) % DEPTH, (u + 1) % 2

                def _next(tt=tt, jn=jn, bn=bn):
                    wait_idx(tt + 1, jn)
                    gather(tt + 1, jn, bn)

                pl.when(tt + 1 < k)(_next)
                pl.when(tt + 3 < k)(
                    lambda tt=tt, u=u: idx_load(tt + 3, (u + 3) % DEPTH))
                wait_gather(tt, j, b)
                scatter(tt, j, b)
            return carry

        lax.fori_loop(0, k // DEPTH, step, 0)
        plsc.subcore_barrier()
        pl.when(s == 0)(
            lambda: pltpu.sync_copy(acc_sh, out_hbm.at[c]))

    return body


def _hprime(x_pad, Wm, deg_parts):
    N_pad, D = x_pad.shape

    def body(x_ref, w_ref, dp_ref, o_ref):
        deg = dp_ref[0, :] + dp_ref[1, :] + 1.0
        dinv = lax.rsqrt(deg)
        h = jnp.dot(x_ref[...], w_ref[...], preferred_element_type=jnp.float32)
        o_ref[...] = h * dinv[:, None]

    return pl.pallas_call(
        body,
        out_shape=jax.ShapeDtypeStruct((N_pad, D), jnp.float32),
    )(x_pad, Wm, deg_parts)


def _finalize(acc, hp, deg_parts, b2, g2, be2, n):
    _, N_pad, D = acc.shape

    def body(acc_ref, hp_ref, dp_ref, b_ref, g_ref, be_ref, o_ref):
        deg = dp_ref[0, :n] + dp_ref[1, :n] + 1.0
        dinv = lax.rsqrt(deg)
        tot = acc_ref[0, :n, :] + acc_ref[1, :n, :] + hp_ref[:n, :]
        pre = tot * dinv[:, None] + b_ref[...]
        mean = jnp.mean(pre, axis=0, keepdims=True)
        var = jnp.mean((pre - mean) ** 2, axis=0, keepdims=True)
        out = (pre - mean) * lax.rsqrt(var + EPS)
        o_ref[...] = jnp.maximum(out * g_ref[...] + be_ref[...], 0.0)

    return pl.pallas_call(
        body,
        out_shape=jax.ShapeDtypeStruct((n, D), jnp.float32),
    )(acc, hp, deg_parts, b2, g2, be2)


def kernel(x, edge_index, W, b, gamma, beta):
    N, D = x.shape
    E = edge_index.shape[1]
    src = edge_index[0].astype(jnp.int32)
    dst = edge_index[1].astype(jnp.int32)

    grain = NW * CHUNK * DEPTH
    E_pad = -(-E // grain) * grain
    N_pad = -(-(N + 1) // (NS * 128)) * (NS * 128)
    epw = E_pad // NW

    n_fill = E_pad - E
    fill = jnp.arange(n_fill, dtype=jnp.int32)
    src_pad = jnp.concatenate([src, fill % N])
    dst_pad = jnp.concatenate([dst, N + fill % (N_pad - N)])
    x_pad = jnp.pad(x, ((0, N_pad - N), (0, 0)))

    dst_l = dst_pad.reshape(NW, epw // CHUNK, 1, CHUNK)
    src_s = src_pad.reshape(E_pad // CHUNK, 1, CHUNK)
    dst_s = dst_pad.reshape(E_pad // CHUNK, 1, CHUNK)

    ones_c = jnp.ones((CHUNK,), jnp.float32)
    zvec = jnp.zeros((N_pad // NS,), jnp.float32)

    deg_parts = _deg_kernel(E_pad, N_pad)(dst_l, ones_c, zvec).reshape(NC, N_pad)
    hp = _hprime(x_pad, W, deg_parts)
    acc = _agg_kernel(E_pad, N_pad, D)(src_s, dst_s, hp)
    return _finalize(acc, hp, deg_parts,
                     b.reshape(1, D), gamma.reshape(1, D), beta.reshape(1, D), N)
